# baseline (speedup 1.0000x reference)
import functools

import jax
import jax.numpy as jnp
from jax import lax
from jax.experimental import pallas as pl
from jax.experimental.pallas import tpu as pltpu
from jax.experimental.pallas import tpu_sc as plsc


@functools.cache
def _build(B, V, D, scale):
    info = plsc.get_sparse_core_info()
    NC, NS, L = info.num_cores, info.num_subcores, info.num_lanes
    NW = NC * NS
    assert B % NW == 0
    bpw = B // NW
    C = 1024
    assert bpw % C == 0 and D % L == 0
    nchunks = bpw // C

    mesh = plsc.VectorSubcoreMesh(core_axis_name="c", subcore_axis_name="s")

    @functools.partial(
        pl.kernel,
        mesh=mesh,
        out_type=jax.ShapeDtypeStruct((B, D), jnp.float32),
        scratch_types=[
            pltpu.VMEM((C,), jnp.int32),
            pltpu.VMEM((C, D), jnp.float32),
            pltpu.SemaphoreType.DMA,
        ],
        compiler_params=pltpu.CompilerParams(use_tc_tiling_on_sc=False),
    )
    def emb(idx_hbm, tbl_hbm, out_hbm, idx_v, rows_v, sem):
        wid = lax.axis_index("s") * NC + lax.axis_index("c")
        base = wid * bpw

        def chunk_body(k, carry):
            off = base + k * C
            pltpu.sync_copy(idx_hbm.at[pl.ds(off, C)], idx_v)
            pltpu.async_copy(tbl_hbm.at[idx_v], rows_v, sem).wait()

            def scale_body(r, carry2):
                for q in range(4):
                    row = r * 4 + q
                    for j in range(D // L):
                        sl = pl.ds(j * L, L)
                        rows_v[row, sl] = rows_v[row, sl] * scale
                return carry2

            lax.fori_loop(0, C // 4, scale_body, 0)
            pltpu.sync_copy(rows_v, out_hbm.at[pl.ds(off, C)])
            return carry

        lax.fori_loop(0, nchunks, chunk_body, 0)

    return emb


def kernel(X, weight):
    B = X.shape[0] * X.shape[1]
    V, D = weight.shape
    scale = float(D) ** 0.5
    idx = X.reshape(B).astype(jnp.int32)
    out = _build(B, V, D, scale)(idx, weight)
    return out.reshape(X.shape[0], X.shape[1], D)


# speedup vs baseline: 1.0582x; 1.0582x over previous
import functools

import jax
import jax.numpy as jnp
from jax import lax
from jax.experimental import pallas as pl
from jax.experimental.pallas import tpu as pltpu
from jax.experimental.pallas import tpu_sc as plsc


@functools.cache
def _build(B, V, D, scale):
    info = plsc.get_sparse_core_info()
    NC, NS, L = info.num_cores, info.num_subcores, info.num_lanes
    NW = NC * NS
    assert B % NW == 0
    bpw = B // NW
    C = 256
    NG = 4
    NSB = 2
    RU = 8
    assert bpw % C == 0 and D % L == 0 and C % RU == 0
    nchunks = bpw // C
    assert nchunks % NG == 0
    kmax = nchunks // NG

    mesh = plsc.VectorSubcoreMesh(core_axis_name="c", subcore_axis_name="s")

    @functools.partial(
        pl.kernel,
        mesh=mesh,
        out_type=jax.ShapeDtypeStruct((B, D), jnp.float32),
        scratch_types=[pltpu.VMEM((C,), jnp.int32)] * NG
        + [pltpu.VMEM((C, D), jnp.float32)] * (NG + NSB)
        + [pltpu.SemaphoreType.DMA] * (NG + NSB),
        compiler_params=pltpu.CompilerParams(use_tc_tiling_on_sc=False),
    )
    def emb(idx_hbm, tbl_hbm, out_hbm, *refs):
        idxs = refs[:NG]
        rows = refs[NG : 2 * NG]
        srows = refs[2 * NG : 2 * NG + NSB]
        gsems = refs[2 * NG + NSB : 3 * NG + NSB]
        ssems = refs[3 * NG + NSB :]
        wid = lax.axis_index("s") * NC + lax.axis_index("c")
        base = wid * bpw

        def gather(c, b):
            pltpu.sync_copy(idx_hbm.at[pl.ds(base + c * C, C)], idxs[b])
            pltpu.async_copy(tbl_hbm.at[idxs[b]], rows[b], gsems[b])

        for b in range(NG):
            gather(b, b)

        def outer(k, carry):
            for j in range(NG):
                s = j % NSB
                c = k * NG + j
                pltpu.make_async_copy(tbl_hbm.at[idxs[j]], rows[j], gsems[j]).wait()
                if j < NSB:

                    @pl.when(k > 0)
                    def _():
                        pltpu.make_async_copy(
                            srows[s], out_hbm.at[pl.ds(base, C)], ssems[s]
                        ).wait()

                else:
                    pltpu.make_async_copy(
                        srows[s], out_hbm.at[pl.ds(base, C)], ssems[s]
                    ).wait()

                def scale_body(r, carry2):
                    row0 = r * RU
                    for q in range(RU):
                        for t in range(D // L):
                            sl = pl.ds(t * L, L)
                            srows[s][row0 + q, sl] = rows[j][row0 + q, sl] * scale
                    return carry2

                lax.fori_loop(0, C // RU, scale_body, 0)

                pltpu.async_copy(srows[s], out_hbm.at[pl.ds(base + c * C, C)], ssems[s])

                @pl.when(k < kmax - 1)
                def _():
                    gather(c + NG, j)

            return carry

        lax.fori_loop(0, kmax, outer, 0)

        for s in range(NSB):
            pltpu.make_async_copy(
                srows[s], out_hbm.at[pl.ds(base, C)], ssems[s]
            ).wait()

    return emb


def kernel(X, weight):
    B = X.shape[0] * X.shape[1]
    V, D = weight.shape
    scale = float(D) ** 0.5
    idx = X.reshape(B).astype(jnp.int32)
    out = _build(B, V, D, scale)(idx, weight)
    return out.reshape(X.shape[0], X.shape[1], D)
